# Initial kernel scaffold; baseline (speedup 1.0000x reference)
import jax
import jax.numpy as jnp
from jax.experimental import pallas as pl


def kernel(ids, table, Wa, ba, Wp, bp, gamma, beta):
    raise NotImplementedError("write your pallas kernel here")


# speedup vs baseline: 3.5573x; 3.5573x over previous
import functools

import jax
import jax.numpy as jnp
from jax import lax
from jax.experimental import pallas as pl
from jax.experimental.pallas import tpu as pltpu
from jax.experimental.pallas import tpu_sc as plsc

V = 100000
D = 64
DM = 512
B, N, T = 1024, 8, 64
BN = B * N
BNT = BN * T

NC, NS = 2, 16
NW = NC * NS
PER_W = BNT // NW
G = 1024


def _sc_gather(ids_flat, table):
    mesh = plsc.VectorSubcoreMesh(core_axis_name="c", subcore_axis_name="s")

    @functools.partial(
        pl.kernel,
        out_type=jax.ShapeDtypeStruct((BNT, D), jnp.float32),
        mesh=mesh,
        scratch_types=[
            pltpu.VMEM((G,), jnp.int32),
            pltpu.VMEM((G, D), jnp.float32),
            pltpu.SemaphoreType.DMA,
        ],
        compiler_params=pltpu.CompilerParams(use_tc_tiling_on_sc=False),
    )
    def gather_kernel(ids_hbm, table_hbm, emb_hbm, idx_v, rows_v, sem):
        wid = lax.axis_index("s") * NC + lax.axis_index("c")
        base = wid * PER_W

        def body(i, carry):
            off = base + i * G
            pltpu.sync_copy(ids_hbm.at[pl.ds(off, G)], idx_v)
            pltpu.async_copy(table_hbm.at[idx_v], rows_v, sem).wait()
            pltpu.sync_copy(rows_v, emb_hbm.at[pl.ds(off, G)])
            return carry

        lax.fori_loop(0, PER_W // G, body, 0)

    return gather_kernel(ids_flat, table)


S = 256


def _tc_body(emb_ref, wa_ref, ba_ref, wp_ref, bp_ref, g_ref, b_ref, out_ref):
    emb = emb_ref[...]
    wa = wa_ref[...]
    logits = jnp.sum(emb * wa[None, :, :], axis=2)
    logits = logits + ba_ref[0, 0]
    logits = jnp.clip(logits, -20.0, 20.0)
    m = jnp.max(logits, axis=1, keepdims=True)
    e = jnp.exp(logits - m)
    w = e / jnp.sum(e, axis=1, keepdims=True)
    pooled = jnp.sum(emb * w[:, :, None], axis=1)
    out = jnp.dot(pooled, wp_ref[...],
                  preferred_element_type=jnp.float32) + bp_ref[...]
    mu = jnp.mean(out, axis=1, keepdims=True)
    var = jnp.mean((out - mu) ** 2, axis=1, keepdims=True)
    y = (out - mu) * lax.rsqrt(var + 1e-5)
    out_ref[...] = y * g_ref[...] + b_ref[...]


def _tc_pool_proj_ln(emb, Wa, ba, Wp, bp, gamma, beta):
    grid = (BN // S,)
    return pl.pallas_call(
        _tc_body,
        grid=grid,
        in_specs=[
            pl.BlockSpec((S, T, D), lambda i: (i, 0, 0)),
            pl.BlockSpec((1, D), lambda i: (0, 0)),
            pl.BlockSpec((1, 1), lambda i: (0, 0)),
            pl.BlockSpec((D, DM), lambda i: (0, 0)),
            pl.BlockSpec((1, DM), lambda i: (0, 0)),
            pl.BlockSpec((1, DM), lambda i: (0, 0)),
            pl.BlockSpec((1, DM), lambda i: (0, 0)),
        ],
        out_specs=pl.BlockSpec((S, DM), lambda i: (i, 0)),
        out_shape=jax.ShapeDtypeStruct((BN, DM), jnp.float32),
    )(emb, Wa.reshape(1, D), ba.reshape(1, 1), Wp, bp.reshape(1, DM),
      gamma.reshape(1, DM), beta.reshape(1, DM))


def kernel(ids, table, Wa, ba, Wp, bp, gamma, beta):
    ids_flat = ids.reshape(BNT).astype(jnp.int32)
    emb = _sc_gather(ids_flat, table)
    out = _tc_pool_proj_ln(emb.reshape(BN, T, D),
                           Wa, ba, Wp, bp, gamma, beta)
    return out.reshape(B, N, DM)


# speedup vs baseline: 8.4278x; 2.3692x over previous
import functools

import jax
import jax.numpy as jnp
from jax import lax
from jax.experimental import pallas as pl
from jax.experimental.pallas import tpu as pltpu
from jax.experimental.pallas import tpu_sc as plsc

V = 100000
D = 64
DM = 512
B, N, T = 1024, 8, 64
BN = B * N
BNT = BN * T
TP = T // 2

NC, NS = 2, 16
NW = NC * NS
PER_W = BNT // NW
G = 1024


def _sc_gather(ids_flat, table):
    mesh = plsc.VectorSubcoreMesh(core_axis_name="c", subcore_axis_name="s")

    @functools.partial(
        pl.kernel,
        out_type=jax.ShapeDtypeStruct((BNT, D), jnp.float32),
        mesh=mesh,
        scratch_types=[
            pltpu.VMEM((G,), jnp.int32),
            pltpu.VMEM((G, D), jnp.float32),
            pltpu.SemaphoreType.DMA,
        ],
        compiler_params=pltpu.CompilerParams(use_tc_tiling_on_sc=False),
    )
    def gather_kernel(ids_hbm, table_hbm, emb_hbm, idx_v, rows_v, sem):
        wid = lax.axis_index("s") * NC + lax.axis_index("c")
        base = wid * PER_W

        def body(i, carry):
            off = base + i * G
            pltpu.sync_copy(ids_hbm.at[pl.ds(off, G)], idx_v)
            pltpu.async_copy(table_hbm.at[idx_v], rows_v, sem).wait()
            pltpu.sync_copy(rows_v, emb_hbm.at[pl.ds(off, G)])
            return carry

        lax.fori_loop(0, PER_W // G, body, 0)

    return gather_kernel(ids_flat, table)


S = 256


def _tc_body(emb_ref, wam_ref, ba_ref, wp_ref, bp_ref, g_ref, b_ref, out_ref):
    e2 = emb_ref[...]
    lfull = jnp.dot(e2, wam_ref[...],
                    preferred_element_type=jnp.float32)
    lfull = jnp.clip(lfull + ba_ref[0, 0], -20.0, 20.0)
    ef = jnp.exp(lfull)
    e3 = e2.reshape(S, TP, 2 * D)
    ef3 = ef.reshape(S, TP, 2 * D)
    pooled_un = jnp.sum(e3 * ef3, axis=1)
    sef = jnp.sum(ef3, axis=1)
    z = sef[:, 0:1] + sef[:, D:D + 1]
    pooled = pooled_un / z
    out = jnp.dot(pooled, wp_ref[...],
                  preferred_element_type=jnp.float32) + bp_ref[...]
    mu = jnp.mean(out, axis=1, keepdims=True)
    var = jnp.mean((out - mu) ** 2, axis=1, keepdims=True)
    y = (out - mu) * lax.rsqrt(var + 1e-5)
    out_ref[...] = y * g_ref[...] + b_ref[...]


def _tc_pool_proj_ln(emb2, WaM, ba, Wp2, bp, gamma, beta):
    grid = (BN // S,)
    return pl.pallas_call(
        _tc_body,
        grid=grid,
        in_specs=[
            pl.BlockSpec((S * TP, 2 * D), lambda i: (i, 0)),
            pl.BlockSpec((2 * D, 2 * D), lambda i: (0, 0)),
            pl.BlockSpec((1, 1), lambda i: (0, 0)),
            pl.BlockSpec((2 * D, DM), lambda i: (0, 0)),
            pl.BlockSpec((1, DM), lambda i: (0, 0)),
            pl.BlockSpec((1, DM), lambda i: (0, 0)),
            pl.BlockSpec((1, DM), lambda i: (0, 0)),
        ],
        out_specs=pl.BlockSpec((S, DM), lambda i: (i, 0)),
        out_shape=jax.ShapeDtypeStruct((BN, DM), jnp.float32),
    )(emb2, WaM, ba.reshape(1, 1), Wp2, bp.reshape(1, DM),
      gamma.reshape(1, DM), beta.reshape(1, DM))


def kernel(ids, table, Wa, ba, Wp, bp, gamma, beta):
    ids_flat = ids.reshape(BNT).astype(jnp.int32)
    emb = _sc_gather(ids_flat, table)
    emb2 = emb.reshape(BNT // 2, 2 * D)
    WaM = jnp.kron(jnp.eye(2, dtype=jnp.float32),
                   jnp.tile(Wa, (1, D)))
    Wp2 = jnp.concatenate([Wp, Wp], axis=0)
    out = _tc_pool_proj_ln(emb2, WaM, ba, Wp2, bp, gamma, beta)
    return out.reshape(B, N, DM)


# speedup vs baseline: 8.8194x; 1.0465x over previous
import functools

import jax
import jax.numpy as jnp
from jax import lax
from jax.experimental import pallas as pl
from jax.experimental.pallas import tpu as pltpu
from jax.experimental.pallas import tpu_sc as plsc

V = 100000
D = 64
DM = 512
B, N, T = 1024, 8, 64
BN = B * N
BNT = BN * T
TP = T // 2

NC, NS = 2, 16
NW = NC * NS
PER_W = BNT // NW
G = 512
NCH = PER_W // G


def _sc_gather(ids_flat, table):
    mesh = plsc.VectorSubcoreMesh(core_axis_name="c", subcore_axis_name="s")

    @functools.partial(
        pl.kernel,
        out_type=jax.ShapeDtypeStruct((BNT, D), jnp.float32),
        mesh=mesh,
        scratch_types=[
            pltpu.VMEM((PER_W,), jnp.int32),
            pltpu.VMEM((G, D), jnp.float32),
            pltpu.VMEM((G, D), jnp.float32),
            pltpu.SemaphoreType.DMA,
            pltpu.SemaphoreType.DMA,
        ],
        compiler_params=pltpu.CompilerParams(use_tc_tiling_on_sc=False),
    )
    def gather_kernel(ids_hbm, table_hbm, emb_hbm, idx_v, rows0, rows1,
                      sem0, sem1):
        wid = lax.axis_index("s") * NC + lax.axis_index("c")
        base = wid * PER_W
        rows = (rows0, rows1)
        sems = (sem0, sem1)

        pltpu.sync_copy(ids_hbm.at[pl.ds(base, PER_W)], idx_v)

        def start_gather(c, b):
            pltpu.async_copy(table_hbm.at[idx_v.at[pl.ds(c * G, G)]],
                             rows[b], sems[b])

        def drain_and_scatter(c, b):
            pltpu.make_async_copy(table_hbm.at[idx_v.at[pl.ds(0, G)]],
                                  rows[b], sems[b]).wait()
            pltpu.sync_copy(rows[b], emb_hbm.at[pl.ds(base + c * G, G)])

        start_gather(0, 0)

        def body(j, carry):
            c0 = 2 * j
            start_gather(c0 + 1, 1)
            drain_and_scatter(c0, 0)
            start_gather(c0 + 2, 0)
            drain_and_scatter(c0 + 1, 1)
            return carry

        lax.fori_loop(0, NCH // 2 - 1, body, 0)
        c0 = NCH - 2
        start_gather(c0 + 1, 1)
        drain_and_scatter(c0, 0)
        drain_and_scatter(c0 + 1, 1)

    return gather_kernel(ids_flat, table)


S = 256


def _tc_body(emb_ref, wam_ref, ba_ref, wp_ref, bp_ref, g_ref, b_ref, out_ref):
    e2 = emb_ref[...]
    lfull = jnp.dot(e2, wam_ref[...],
                    preferred_element_type=jnp.float32)
    lfull = jnp.clip(lfull + ba_ref[0, 0], -20.0, 20.0)
    ef = jnp.exp(lfull)
    e3 = e2.reshape(S, TP, 2 * D)
    ef3 = ef.reshape(S, TP, 2 * D)
    pooled_un = jnp.sum(e3 * ef3, axis=1)
    sef = jnp.sum(ef3, axis=1)
    z = sef[:, 0:1] + sef[:, D:D + 1]
    pooled = pooled_un / z
    out = jnp.dot(pooled, wp_ref[...],
                  preferred_element_type=jnp.float32) + bp_ref[...]
    mu = jnp.mean(out, axis=1, keepdims=True)
    var = jnp.mean((out - mu) ** 2, axis=1, keepdims=True)
    y = (out - mu) * lax.rsqrt(var + 1e-5)
    out_ref[...] = y * g_ref[...] + b_ref[...]


def _tc_pool_proj_ln(emb2, WaM, ba, Wp2, bp, gamma, beta):
    grid = (BN // S,)
    return pl.pallas_call(
        _tc_body,
        grid=grid,
        in_specs=[
            pl.BlockSpec((S * TP, 2 * D), lambda i: (i, 0)),
            pl.BlockSpec((2 * D, 2 * D), lambda i: (0, 0)),
            pl.BlockSpec((1, 1), lambda i: (0, 0)),
            pl.BlockSpec((2 * D, DM), lambda i: (0, 0)),
            pl.BlockSpec((1, DM), lambda i: (0, 0)),
            pl.BlockSpec((1, DM), lambda i: (0, 0)),
            pl.BlockSpec((1, DM), lambda i: (0, 0)),
        ],
        out_specs=pl.BlockSpec((S, DM), lambda i: (i, 0)),
        out_shape=jax.ShapeDtypeStruct((BN, DM), jnp.float32),
    )(emb2, WaM, ba.reshape(1, 1), Wp2, bp.reshape(1, DM),
      gamma.reshape(1, DM), beta.reshape(1, DM))


def kernel(ids, table, Wa, ba, Wp, bp, gamma, beta):
    ids_flat = ids.reshape(BNT).astype(jnp.int32)
    emb = _sc_gather(ids_flat, table)
    emb2 = emb.reshape(BNT // 2, 2 * D)
    WaM = jnp.kron(jnp.eye(2, dtype=jnp.float32),
                   jnp.tile(Wa, (1, D)))
    Wp2 = jnp.concatenate([Wp, Wp], axis=0)
    out = _tc_pool_proj_ln(emb2, WaM, ba, Wp2, bp, gamma, beta)
    return out.reshape(B, N, DM)
